# baseline (speedup 1.0000x reference)
import functools

import jax
import jax.numpy as jnp
from jax import lax
from jax.experimental import pallas as pl
from jax.experimental.pallas import tpu as pltpu
from jax.experimental.pallas import tpu_sc as plsc

N = 10000
E = 160000
D = 256
R = 50
NBASES = 10
NCLS = 40
HALF = 128
NTILES = 16
EPT = E // 2 // NTILES
PADV = 40
EPAD = EPT + PADV
CH = 80
EB = 1680
NEB = EPAD // EB
NCHB = EB // CH
NBUF = 3
DUMP = N
ZCH = 200
NAGG = 10200
NZ = NAGG // ZCH
NW = N // ZCH
NBLK = 10
BN = N // NBLK
RGRP = 10


def _w_body(coef_ref, basis_ref, out_ref):
    for g in range(RGRP):
        w = coef_ref[g, 0, 0] * basis_ref[0]
        for b in range(1, NBASES):
            w = w + coef_ref[g, 0, b] * basis_ref[b]
        out_ref[g] = w.astype(jnp.bfloat16)


def _build_w(coef, basis):
    return pl.pallas_call(
        _w_body,
        grid=(R // RGRP,),
        in_specs=[
            pl.BlockSpec((RGRP, 1, NBASES), lambda r: (r, 0, 0)),
            pl.BlockSpec((NBASES, D, D), lambda r: (0, 0, 0)),
        ],
        out_specs=pl.BlockSpec((RGRP, D, D), lambda r: (r, 0, 0)),
        out_shape=jax.ShapeDtypeStruct((R, D, D), jnp.bfloat16),
    )(coef.reshape(R, 1, NBASES), basis)


def _ht_body(w_ref, x_ref, out_ref):
    out_ref[0] = jnp.dot(x_ref[...], w_ref[0],
                         preferred_element_type=jnp.float32)


def _build_ht_half(x16, w_all, h):
    return pl.pallas_call(
        _ht_body,
        grid=(R,),
        in_specs=[
            pl.BlockSpec((1, D, HALF), lambda r: (r, 0, h)),
            pl.BlockSpec((N, D), lambda r: (0, 0)),
        ],
        out_specs=pl.BlockSpec((1, N, HALF), lambda r: (r, 0, 0)),
        out_shape=jax.ShapeDtypeStruct((R, N, HALF), jnp.float32),
    )(w_all, x16)


def _edge_kernel_body(ht_hbm, src_hbm, typ_hbm, dst_hbm, zeros_hbm, out_hbm,
                      src_buf, typ_buf, dst_buf,
                      idx0, idx1, idx2, gb0, gb1, gb2, agg,
                      gsem0, gsem1, gsem2, ssem0, ssem1, ssem2):
    c = lax.axis_index("c")
    s = lax.axis_index("s")
    idx_bufs = (idx0, idx1, idx2)
    gbufs = (gb0, gb1, gb2)
    gsems = (gsem0, gsem1, gsem2)
    ssems = (ssem0, ssem1, ssem2)

    for k in range((NZ + NTILES - 1) // NTILES):
        ch = s + k * NTILES

        @pl.when(ch < NZ)
        def _():
            pltpu.sync_copy(zeros_hbm, agg.at[pl.ds(ch * ZCH, ZCH)])
    plsc.subcore_barrier()

    def _mk_idx(j, k):
        eoff = k * CH
        for v in range(CH // 16):
            t = typ_buf[pl.ds(eoff + v * 16, 16)]
            sr = src_buf[pl.ds(eoff + v * 16, 16)]
            idx_bufs[j][pl.ds(v * 16, 16)] = t * N + sr

    def _gather_start(j):
        pltpu.async_copy(ht_hbm.at[idx_bufs[j]], gbufs[j], gsems[j])

    def _visit(j, k):
        pltpu.make_async_copy(ht_hbm.at[idx_bufs[j]], gbufs[j],
                              gsems[j]).wait()
        pltpu.async_copy(gbufs[j], agg.at[dst_buf.at[k]], ssems[j], add=True)
        kn = k + NBUF

        @pl.when(kn < NCHB)
        def _():
            _mk_idx(j, kn)
        pltpu.make_async_copy(gbufs[j], agg.at[dst_buf.at[k]],
                              ssems[j]).wait()

        @pl.when(kn < NCHB)
        def _():
            _gather_start(j)

    def _block(b, carry):
        base = (c * NTILES + s) * EPAD + b * EB
        pltpu.sync_copy(src_hbm.at[pl.ds(base, EB)], src_buf)
        pltpu.sync_copy(typ_hbm.at[pl.ds(base, EB)], typ_buf)
        pltpu.sync_copy(dst_hbm.at[c * NTILES + s, b], dst_buf)

        for j in range(NBUF):
            _mk_idx(j, j)
            _gather_start(j)

        nround = NCHB // NBUF - 1

        def _round(o, cr):
            for j in range(NBUF):
                _visit(j, o * NBUF + j)
            return cr
        lax.fori_loop(0, nround, _round, 0)
        for k in range(nround * NBUF, NCHB):
            _visit(k % NBUF, k)
        return carry
    lax.fori_loop(0, NEB, _block, 0)

    plsc.subcore_barrier()
    for k in range((NW + NTILES - 1) // NTILES):
        ch = s + k * NTILES

        @pl.when(ch < NW)
        def _():
            pltpu.sync_copy(agg.at[pl.ds(ch * ZCH, ZCH)],
                            out_hbm.at[c * NW + ch])


def _edge_aggregate(ht2, srcf, typf, dst4, zeros):
    mesh = plsc.VectorSubcoreMesh(core_axis_name="c", subcore_axis_name="s")
    k = functools.partial(
        pl.kernel,
        mesh=mesh,
        out_type=jax.ShapeDtypeStruct((2 * NW, ZCH, HALF), jnp.float32),
        scratch_types=[
            pltpu.VMEM((EB,), jnp.int32),
            pltpu.VMEM((EB,), jnp.int32),
            pltpu.VMEM((NCHB, CH), jnp.int32),
            pltpu.VMEM((CH,), jnp.int32),
            pltpu.VMEM((CH,), jnp.int32),
            pltpu.VMEM((CH,), jnp.int32),
            pltpu.VMEM((CH, HALF), jnp.float32),
            pltpu.VMEM((CH, HALF), jnp.float32),
            pltpu.VMEM((CH, HALF), jnp.float32),
            pltpu.VMEM_SHARED((NAGG, HALF), jnp.float32),
            pltpu.SemaphoreType.DMA,
            pltpu.SemaphoreType.DMA,
            pltpu.SemaphoreType.DMA,
            pltpu.SemaphoreType.DMA,
            pltpu.SemaphoreType.DMA,
            pltpu.SemaphoreType.DMA,
        ],
    )(_edge_kernel_body)
    return k(ht2, srcf, typf, dst4, zeros)


def _fin_body(aggl_ref, aggr_ref, x_ref, wr_ref, bias_ref, w1_ref, b1_ref,
              w2_ref, b2_ref, out_ref, acc_ref):
    i = pl.program_id(0)
    xr = jnp.dot(x_ref[...], wr_ref[...], preferred_element_type=jnp.float32,
                 precision=lax.Precision.HIGHEST)
    a0 = aggl_ref[0] + aggl_ref[1]
    a1 = aggr_ref[0] + aggr_ref[1]
    h0 = jnp.maximum(a0 + xr[:, :HALF] + bias_ref[:, :HALF], 0.0)
    h1 = jnp.maximum(a1 + xr[:, HALF:] + bias_ref[:, HALF:], 0.0)
    sums = jnp.concatenate(
        [jnp.sum(h0, axis=0, keepdims=True), jnp.sum(h1, axis=0, keepdims=True)],
        axis=1)

    @pl.when(i == 0)
    def _():
        acc_ref[...] = sums

    @pl.when(i > 0)
    def _():
        acc_ref[...] = acc_ref[...] + sums

    @pl.when(i == NBLK - 1)
    def _():
        pooled = acc_ref[...] * (1.0 / N)
        z = jnp.maximum(
            jnp.dot(pooled, w1_ref[...], preferred_element_type=jnp.float32,
                    precision=lax.Precision.HIGHEST) + b1_ref[...], 0.0)
        out_ref[...] = jnp.dot(z, w2_ref[...],
                               preferred_element_type=jnp.float32,
                               precision=lax.Precision.HIGHEST) + b2_ref[...]


def _finish(aggl, aggr, x, W_root, bias, W1, b1, W2, b2):
    return pl.pallas_call(
        _fin_body,
        grid=(NBLK,),
        in_specs=[
            pl.BlockSpec((2, BN, HALF), lambda i: (0, i, 0)),
            pl.BlockSpec((2, BN, HALF), lambda i: (0, i, 0)),
            pl.BlockSpec((BN, D), lambda i: (i, 0)),
            pl.BlockSpec((D, D), lambda i: (0, 0)),
            pl.BlockSpec((1, D), lambda i: (0, 0)),
            pl.BlockSpec((D, 16), lambda i: (0, 0)),
            pl.BlockSpec((1, 16), lambda i: (0, 0)),
            pl.BlockSpec((16, NCLS), lambda i: (0, 0)),
            pl.BlockSpec((1, NCLS), lambda i: (0, 0)),
        ],
        out_specs=pl.BlockSpec((1, NCLS), lambda i: (0, 0)),
        out_shape=jax.ShapeDtypeStruct((1, NCLS), jnp.float32),
        scratch_shapes=[pltpu.VMEM((1, D), jnp.float32)],
    )(aggl, aggr, x, W_root, bias, W1, b1, W2, b2)


def kernel(x, edge_index, edge_type, basis, coef, W_root, bias, W1, b1, W2,
           b2):
    src = edge_index[0].astype(jnp.int32)
    dst = edge_index[1].astype(jnp.int32)
    typ = edge_type.astype(jnp.int32)

    pad3 = ((0, 0), (0, 0), (0, PADV))
    srcf = jnp.pad(src.reshape(2, NTILES, EPT), pad3).reshape(-1)
    typf = jnp.pad(typ.reshape(2, NTILES, EPT), pad3).reshape(-1)
    dst4 = jnp.pad(dst.reshape(2, NTILES, EPT), pad3,
                   constant_values=DUMP).reshape(2 * NTILES, NEB, NCHB, CH)
    zeros = jnp.zeros((ZCH, HALF), jnp.float32)

    w_all = _build_w(coef, basis)
    x16 = x.astype(jnp.bfloat16)
    ht_l = _build_ht_half(x16, w_all, 0).reshape(R * N, HALF)
    aggl = _edge_aggregate(ht_l, srcf, typf, dst4, zeros)
    ht_r = _build_ht_half(x16, w_all, 1).reshape(R * N, HALF)
    aggr = _edge_aggregate(ht_r, srcf, typf, dst4, zeros)

    out = _finish(aggl.reshape(2, N, HALF), aggr.reshape(2, N, HALF),
                  x, W_root, bias.reshape(1, D), W1,
                  b1.reshape(1, 16), W2, b2.reshape(1, NCLS))
    return out.reshape(NCLS)


# speedup vs baseline: 1.3679x; 1.3679x over previous
import functools

import jax
import jax.numpy as jnp
from jax import lax
from jax.experimental import pallas as pl
from jax.experimental.pallas import tpu as pltpu
from jax.experimental.pallas import tpu_sc as plsc

N = 10000
E = 160000
D = 256
R = 50
NBASES = 10
NCLS = 40
HALF = 128
NTILES = 16
EPT = E // NTILES
CH = 80
EB = 2000
NEB = EPT // EB
NCHB = EB // CH
NBUF = 3
ZCH = 200
NZCH = N // ZCH
NBLK = 10
BN = N // NBLK
NBLKA = 1
BNA = N // NBLKA


RGRP = 10


def _w_body(coef_ref, basis_ref, out_ref):
    for g in range(RGRP):
        w = coef_ref[g, 0, 0] * basis_ref[0]
        for b in range(1, NBASES):
            w = w + coef_ref[g, 0, b] * basis_ref[b]
        out_ref[g] = w.astype(jnp.bfloat16)


def _build_w(coef, basis):
    return pl.pallas_call(
        _w_body,
        grid=(R // RGRP,),
        in_specs=[
            pl.BlockSpec((RGRP, 1, NBASES), lambda r: (r, 0, 0)),
            pl.BlockSpec((NBASES, D, D), lambda r: (0, 0, 0)),
        ],
        out_specs=pl.BlockSpec((RGRP, D, D), lambda r: (r, 0, 0)),
        out_shape=jax.ShapeDtypeStruct((R, D, D), jnp.bfloat16),
    )(coef.reshape(R, 1, NBASES), basis)


def _ht_body(w_ref, x_ref, out_ref):
    h = jnp.dot(x_ref[...], w_ref[0], preferred_element_type=jnp.float32)
    out_ref[0, 0] = h[:, :HALF]
    out_ref[1, 0] = h[:, HALF:]


def _build_ht(x16, w_all):
    return pl.pallas_call(
        _ht_body,
        grid=(NBLKA, R),
        in_specs=[
            pl.BlockSpec((1, D, D), lambda j, r: (r, 0, 0)),
            pl.BlockSpec((BNA, D), lambda j, r: (j, 0)),
        ],
        out_specs=pl.BlockSpec((2, 1, BNA, HALF), lambda j, r: (0, r, j, 0)),
        out_shape=jax.ShapeDtypeStruct((2, R, N, HALF), jnp.float32),
    )(w_all, x16)


def _edge_kernel_body(ht_hbm, src_hbm, typ_hbm, dst_hbm, zeros_hbm, out_hbm,
                      src_buf, typ_buf, dst_buf,
                      idx0, idx1, idx2, gb0, gb1, gb2, agg,
                      gsem0, gsem1, gsem2, ssem0, ssem1, ssem2):
    c = lax.axis_index("c")
    s = lax.axis_index("s")
    idx_bufs = (idx0, idx1, idx2)
    gbufs = (gb0, gb1, gb2)
    gsems = (gsem0, gsem1, gsem2)
    ssems = (ssem0, ssem1, ssem2)

    for k in range((NZCH + NTILES - 1) // NTILES):
        ch = s + k * NTILES

        @pl.when(ch < NZCH)
        def _():
            pltpu.sync_copy(zeros_hbm, agg.at[pl.ds(ch * ZCH, ZCH)])
    plsc.subcore_barrier()

    def _mk_idx(j, k):
        eoff = k * CH
        for v in range(CH // 16):
            t = typ_buf[pl.ds(eoff + v * 16, 16)]
            sr = src_buf[pl.ds(eoff + v * 16, 16)]
            idx_bufs[j][pl.ds(v * 16, 16)] = t * N + sr + c * (R * N)

    def _gather_start(j):
        pltpu.async_copy(ht_hbm.at[idx_bufs[j]], gbufs[j], gsems[j])

    def _visit(j, k):
        pltpu.make_async_copy(ht_hbm.at[idx_bufs[j]], gbufs[j],
                              gsems[j]).wait()
        pltpu.async_copy(gbufs[j], agg.at[dst_buf.at[k]], ssems[j], add=True)
        kn = k + NBUF

        @pl.when(kn < NCHB)
        def _():
            _mk_idx(j, kn)
        pltpu.make_async_copy(gbufs[j], agg.at[dst_buf.at[k]],
                              ssems[j]).wait()

        @pl.when(kn < NCHB)
        def _():
            _gather_start(j)

    def _block(b, carry):
        base = s * EPT + b * EB
        pltpu.sync_copy(src_hbm.at[pl.ds(base, EB)], src_buf)
        pltpu.sync_copy(typ_hbm.at[pl.ds(base, EB)], typ_buf)
        pltpu.sync_copy(dst_hbm.at[s, b], dst_buf)

        for j in range(NBUF):
            _mk_idx(j, j)
            _gather_start(j)

        nround = NCHB // NBUF - 1

        def _round(o, cr):
            for j in range(NBUF):
                _visit(j, o * NBUF + j)
            return cr
        lax.fori_loop(0, nround, _round, 0)
        for k in range(nround * NBUF, NCHB):
            _visit(k % NBUF, k)
        return carry
    lax.fori_loop(0, NEB, _block, 0)

    plsc.subcore_barrier()
    for k in range((NZCH + NTILES - 1) // NTILES):
        ch = s + k * NTILES

        @pl.when(ch < NZCH)
        def _():
            pltpu.sync_copy(agg.at[pl.ds(ch * ZCH, ZCH)],
                            out_hbm.at[c * NZCH + ch])


def _edge_aggregate(ht2, src, typ, dst4, zeros):
    mesh = plsc.VectorSubcoreMesh(core_axis_name="c", subcore_axis_name="s")
    k = functools.partial(
        pl.kernel,
        mesh=mesh,
        out_type=jax.ShapeDtypeStruct((2 * NZCH, ZCH, HALF), jnp.float32),
        scratch_types=[
            pltpu.VMEM((EB,), jnp.int32),
            pltpu.VMEM((EB,), jnp.int32),
            pltpu.VMEM((NCHB, CH), jnp.int32),
            pltpu.VMEM((CH,), jnp.int32),
            pltpu.VMEM((CH,), jnp.int32),
            pltpu.VMEM((CH,), jnp.int32),
            pltpu.VMEM((CH, HALF), jnp.float32),
            pltpu.VMEM((CH, HALF), jnp.float32),
            pltpu.VMEM((CH, HALF), jnp.float32),
            pltpu.VMEM_SHARED((N, HALF), jnp.float32),
            pltpu.SemaphoreType.DMA,
            pltpu.SemaphoreType.DMA,
            pltpu.SemaphoreType.DMA,
            pltpu.SemaphoreType.DMA,
            pltpu.SemaphoreType.DMA,
            pltpu.SemaphoreType.DMA,
        ],
    )(_edge_kernel_body)
    return k(ht2, src, typ, dst4, zeros)


def _fin_body(agg_ref, x_ref, wr_ref, bias_ref, w1_ref, b1_ref, w2_ref,
              b2_ref, out_ref, acc_ref):
    i = pl.program_id(0)
    xr = jnp.dot(x_ref[...], wr_ref[...], preferred_element_type=jnp.float32,
                 precision=lax.Precision.HIGHEST)
    h0 = jnp.maximum(agg_ref[0] + xr[:, :HALF] + bias_ref[:, :HALF], 0.0)
    h1 = jnp.maximum(agg_ref[1] + xr[:, HALF:] + bias_ref[:, HALF:], 0.0)
    sums = jnp.concatenate(
        [jnp.sum(h0, axis=0, keepdims=True), jnp.sum(h1, axis=0, keepdims=True)],
        axis=1)

    @pl.when(i == 0)
    def _():
        acc_ref[...] = sums

    @pl.when(i > 0)
    def _():
        acc_ref[...] = acc_ref[...] + sums

    @pl.when(i == NBLK - 1)
    def _():
        pooled = acc_ref[...] * (1.0 / N)
        z = jnp.maximum(
            jnp.dot(pooled, w1_ref[...], preferred_element_type=jnp.float32,
                    precision=lax.Precision.HIGHEST) + b1_ref[...], 0.0)
        out_ref[...] = jnp.dot(z, w2_ref[...],
                               preferred_element_type=jnp.float32,
                               precision=lax.Precision.HIGHEST) + b2_ref[...]


def _finish(agg2, x, W_root, bias, W1, b1, W2, b2):
    return pl.pallas_call(
        _fin_body,
        grid=(NBLK,),
        in_specs=[
            pl.BlockSpec((2, BN, HALF), lambda i: (0, i, 0)),
            pl.BlockSpec((BN, D), lambda i: (i, 0)),
            pl.BlockSpec((D, D), lambda i: (0, 0)),
            pl.BlockSpec((1, D), lambda i: (0, 0)),
            pl.BlockSpec((D, 16), lambda i: (0, 0)),
            pl.BlockSpec((1, 16), lambda i: (0, 0)),
            pl.BlockSpec((16, NCLS), lambda i: (0, 0)),
            pl.BlockSpec((1, NCLS), lambda i: (0, 0)),
        ],
        out_specs=pl.BlockSpec((1, NCLS), lambda i: (0, 0)),
        out_shape=jax.ShapeDtypeStruct((1, NCLS), jnp.float32),
        scratch_shapes=[pltpu.VMEM((1, D), jnp.float32)],
    )(agg2, x, W_root, bias, W1, b1, W2, b2)


def kernel(x, edge_index, edge_type, basis, coef, W_root, bias, W1, b1, W2,
           b2):
    src = edge_index[0].astype(jnp.int32)
    dst = edge_index[1].astype(jnp.int32)
    typ = edge_type.astype(jnp.int32)

    w_all = _build_w(coef, basis)
    ht = _build_ht(x.astype(jnp.bfloat16), w_all)
    ht2 = ht.reshape(2 * R * N, HALF)
    dst4 = dst.reshape(NTILES, NEB, NCHB, CH)
    zeros = jnp.zeros((ZCH, HALF), jnp.float32)
    agg2 = _edge_aggregate(ht2, src, typ, dst4, zeros).reshape(2, N, HALF)
    out = _finish(agg2, x, W_root, bias.reshape(1, D), W1,
                  b1.reshape(1, 16), W2, b2.reshape(1, NCLS))
    return out.reshape(NCLS)
